# baseline (speedup 1.0000x reference)
import jax
import jax.numpy as jnp
from jax.experimental import pallas as pl
from jax.experimental.pallas import tpu as pltpu

_BM = 400


def _body(x_ref, fcw_ref, w_ref, g_ref, b_ref, o_ref, x2_s, y_s, sum_s, ss_s):
    i = pl.program_id(0)
    nb = pl.num_programs(0)

    @pl.when(i == 0)
    def _prologue():
        x2_s[...] = jax.lax.dot_general(
            x_ref[...], fcw_ref[...],
            dimension_numbers=(((1,), (1,)), ((), ())),
            preferred_element_type=jnp.float32,
        ).astype(jnp.bfloat16)
        sum_s[...] = jnp.zeros_like(sum_s)
        ss_s[...] = jnp.zeros_like(ss_s)

    yb = jnp.dot(w_ref[...].astype(jnp.bfloat16), x2_s[...],
                 preferred_element_type=jnp.float32)
    y_s[pl.ds(i * _BM, _BM), :] = yb
    sum_s[...] += jnp.sum(yb, axis=0, keepdims=True)
    ss_s[...] += jnp.sum(yb * yb, axis=0, keepdims=True)

    @pl.when(i == nb - 1)
    def _epilogue():
        n = y_s.shape[0]
        mean = sum_s[...] * (1.0 / n)
        var = ss_s[...] * (1.0 / n) - mean * mean
        scale = g_ref[...] * jax.lax.rsqrt(var + 1e-5)
        shift = b_ref[...] - mean * scale
        o_ref[...] = y_s[...] * scale + shift


def kernel(x, W, fc_w, fc_b, bn_gamma, bn_beta):
    del fc_b
    n, d_in = x.shape
    d_out = fc_w.shape[0]
    nb = n // _BM

    return pl.pallas_call(
        _body,
        grid=(nb,),
        in_specs=[
            pl.BlockSpec((n, d_in), lambda i: (0, 0)),
            pl.BlockSpec((d_out, d_in), lambda i: (0, 0)),
            pl.BlockSpec((_BM, n), lambda i: (i, 0)),
            pl.BlockSpec((1, d_out), lambda i: (0, 0)),
            pl.BlockSpec((1, d_out), lambda i: (0, 0)),
        ],
        out_specs=pl.BlockSpec((n, d_out), lambda i: (0, 0)),
        out_shape=jax.ShapeDtypeStruct((n, d_out), jnp.float32),
        scratch_shapes=[
            pltpu.VMEM((n, d_out), jnp.bfloat16),
            pltpu.VMEM((n, d_out), jnp.float32),
            pltpu.VMEM((1, d_out), jnp.float32),
            pltpu.VMEM((1, d_out), jnp.float32),
        ],
    )(x, fc_w, W, bn_gamma.reshape(1, d_out), bn_beta.reshape(1, d_out))


# speedup vs baseline: 1.0028x; 1.0028x over previous
import jax
import jax.numpy as jnp
from jax.experimental import pallas as pl
from jax.experimental.pallas import tpu as pltpu

_BM = 400


def _body(x_ref, fcw_ref, w_ref, g_ref, b_ref, o_ref, x2_s, sum_s, ss_s):
    i = pl.program_id(0)
    nb = pl.num_programs(0)

    @pl.when(i == 0)
    def _prologue():
        x2_s[...] = jax.lax.dot_general(
            x_ref[...], fcw_ref[...],
            dimension_numbers=(((1,), (1,)), ((), ())),
            preferred_element_type=jnp.float32,
        ).astype(jnp.bfloat16)
        sum_s[...] = jnp.zeros_like(sum_s)
        ss_s[...] = jnp.zeros_like(ss_s)

    yb = jnp.dot(w_ref[...].astype(jnp.bfloat16), x2_s[...],
                 preferred_element_type=jnp.float32)
    o_ref[pl.ds(i * _BM, _BM), :] = yb
    sum_s[...] += jnp.sum(yb, axis=0, keepdims=True)
    ss_s[...] += jnp.sum(yb * yb, axis=0, keepdims=True)

    @pl.when(i == nb - 1)
    def _epilogue():
        n = o_ref.shape[0]
        mean = sum_s[...] * (1.0 / n)
        var = ss_s[...] * (1.0 / n) - mean * mean
        scale = g_ref[...] * jax.lax.rsqrt(var + 1e-5)
        shift = b_ref[...] - mean * scale
        o_ref[...] = o_ref[...] * scale + shift


def kernel(x, W, fc_w, fc_b, bn_gamma, bn_beta):
    del fc_b
    n, d_in = x.shape
    d_out = fc_w.shape[0]
    nb = n // _BM

    return pl.pallas_call(
        _body,
        grid=(nb,),
        in_specs=[
            pl.BlockSpec((n, d_in), lambda i: (0, 0)),
            pl.BlockSpec((d_out, d_in), lambda i: (0, 0)),
            pl.BlockSpec((_BM, n), lambda i: (i, 0)),
            pl.BlockSpec((1, d_out), lambda i: (0, 0)),
            pl.BlockSpec((1, d_out), lambda i: (0, 0)),
        ],
        out_specs=pl.BlockSpec((n, d_out), lambda i: (0, 0)),
        out_shape=jax.ShapeDtypeStruct((n, d_out), jnp.float32),
        scratch_shapes=[
            pltpu.VMEM((n, d_out), jnp.bfloat16),
            pltpu.VMEM((1, d_out), jnp.float32),
            pltpu.VMEM((1, d_out), jnp.float32),
        ],
    )(x, fc_w, W, bn_gamma.reshape(1, d_out), bn_beta.reshape(1, d_out))
